# Initial kernel scaffold; baseline (speedup 1.0000x reference)
import functools

import jax
import jax.numpy as jnp
from jax import lax
from jax.experimental import pallas as pl
from jax.experimental.pallas import tpu as pltpu
from jax.experimental.pallas import tpu_sc as plsc

_ND = 13
_NF = 26
_VOCAB = 100000
_EMB = 16
_ROW = _ND + _NF * _EMB

_NC = 2
_NS = 16
_NW = _NC * _NS

_T = 128
_IDX_PER_CHUNK = _T * _NF
_IDX_ROWS = _IDX_PER_CHUNK // 128
_VOUT = _T * _ROW


def _sc_embed(n_tokens):
    chunks_per_tile = n_tokens // (_NW * _T)
    mesh = plsc.VectorSubcoreMesh(core_axis_name="c", subcore_axis_name="s")

    @functools.partial(
        pl.kernel,
        mesh=mesh,
        out_type=jax.ShapeDtypeStruct((n_tokens * _ROW,), jnp.float32),
        scratch_types=[
            pltpu.VMEM((_IDX_ROWS, 128), jnp.int32),
            pltpu.VMEM((_IDX_PER_CHUNK, _EMB), jnp.float32),
            pltpu.VMEM((_T, 16), jnp.float32),
            pltpu.VMEM((_VOUT,), jnp.float32),
            pltpu.SemaphoreType.DMA,
        ],
    )
    def k(ftab, idx2, xdp, out, idx_v, emb_v, dense_v, v_out, sem):
        wid = lax.axis_index("s") * _NC + lax.axis_index("c")
        lanes = lax.iota(jnp.int32, 16)
        dmask = lanes < _ND

        def chunk(c, carry):
            g = wid * chunks_per_tile + c
            tok0 = g * _T
            pltpu.sync_copy(idx2.at[pl.ds(g * _IDX_ROWS, _IDX_ROWS)], idx_v)
            pltpu.sync_copy(xdp.at[pl.ds(tok0, _T)], dense_v)
            cps = [
                pltpu.async_copy(
                    ftab.at[idx_v.at[j]], emb_v.at[pl.ds(j * 128, 128)], sem)
                for j in range(_IDX_ROWS)
            ]
            for cp in cps:
                cp.wait()

            def tok(t, carry2):
                off = t * _ROW + lanes
                plsc.store_scatter(v_out, [off], dense_v[t], mask=dmask)
                for f in range(_NF):
                    plsc.store_scatter(
                        v_out, [off + (_ND + f * _EMB)], emb_v[t * _NF + f])
                return carry2

            lax.fori_loop(0, _T, tok, 0)
            pltpu.sync_copy(v_out, out.at[pl.ds(tok0 * _ROW, _VOUT)])
            return carry

        lax.fori_loop(0, chunks_per_tile, chunk, 0)

    return k


def kernel(x, tables):
    b, seq, f_dim = x.shape
    n = b * seq
    xf = x.reshape(n, f_dim)
    idx = xf[:, _ND:].astype(jnp.int32) + (
        jnp.arange(_NF, dtype=jnp.int32) * _VOCAB)[None, :]
    idx2 = idx.reshape(n * _NF // 128, 128)
    xdp = jnp.pad(xf[:, :_ND], ((0, 0), (0, 16 - _ND)))
    ftab = tables.reshape(_NF * _VOCAB, _EMB)
    out = _sc_embed(n)(ftab, idx2, xdp)
    return out.reshape(b, seq, _ROW)


# speedup vs baseline: 5.8763x; 5.8763x over previous
import functools

import jax
import jax.numpy as jnp
from jax import lax
from jax.experimental import pallas as pl
from jax.experimental.pallas import tpu as pltpu
from jax.experimental.pallas import tpu_sc as plsc

_ND = 13
_NF = 26
_VOCAB = 100000
_EMB = 16
_ROW = _ND + _NF * _EMB

_NC = 2
_NS = 16
_NW = _NC * _NS

_T = 128
_IDX_PER_CHUNK = _T * _NF
_IDX_ROWS = _IDX_PER_CHUNK // 128
_VOUT = _T * _ROW


def _sc_embed(n_tokens):
    chunks_per_tile = n_tokens // (_NW * _T)
    mesh = plsc.VectorSubcoreMesh(core_axis_name="c", subcore_axis_name="s")

    @functools.partial(
        pl.kernel,
        mesh=mesh,
        out_type=jax.ShapeDtypeStruct((n_tokens * _ROW,), jnp.float32),
        scratch_types=[
            pltpu.VMEM((_IDX_ROWS, 128), jnp.int32),
            pltpu.VMEM((_IDX_PER_CHUNK, _EMB), jnp.float32),
            pltpu.VMEM((_T, 16), jnp.float32),
            pltpu.VMEM((_VOUT,), jnp.float32),
            pltpu.SemaphoreType.DMA,
        ],
        compiler_params=pltpu.CompilerParams(
            use_tc_tiling_on_sc=False, needs_layout_passes=False),
    )
    def k(ftab, idx2, xdp, out, idx_v, emb_v, dense_v, v_out, sem):
        wid = lax.axis_index("s") * _NC + lax.axis_index("c")
        lanes = lax.iota(jnp.int32, 16)
        dmask = lanes < _ND

        def chunk(c, carry):
            g = wid * chunks_per_tile + c
            tok0 = g * _T
            pltpu.sync_copy(idx2.at[pl.ds(g * _IDX_ROWS, _IDX_ROWS)], idx_v)
            pltpu.sync_copy(xdp.at[pl.ds(tok0, _T)], dense_v)
            cps = [
                pltpu.async_copy(
                    ftab.at[idx_v.at[j]], emb_v.at[pl.ds(j * 128, 128)], sem)
                for j in range(_IDX_ROWS)
            ]
            for cp in cps:
                cp.wait()

            def tok(t, carry2):
                off = t * _ROW + lanes
                plsc.store_scatter(v_out, [off], dense_v[t], mask=dmask)
                for f in range(_NF):
                    plsc.store_scatter(
                        v_out, [off + (_ND + f * _EMB)], emb_v[t * _NF + f])
                return carry2

            lax.fori_loop(0, _T, tok, 0)
            pltpu.sync_copy(v_out, out.at[pl.ds(tok0 * _ROW, _VOUT)])
            return carry

        lax.fori_loop(0, chunks_per_tile, chunk, 0)

    return k


def kernel(x, tables):
    b, seq, f_dim = x.shape
    n = b * seq
    xf = x.reshape(n, f_dim)
    idx = xf[:, _ND:].astype(jnp.int32) + (
        jnp.arange(_NF, dtype=jnp.int32) * _VOCAB)[None, :]
    idx2 = idx.reshape(n * _NF // 128, 128)
    xdp = jnp.pad(xf[:, :_ND], ((0, 0), (0, 16 - _ND)))
    ftab = tables.reshape(_NF * _VOCAB, _EMB)
    out = _sc_embed(n)(ftab, idx2, xdp)
    return out.reshape(b, seq, _ROW)


# speedup vs baseline: 6.7010x; 1.1403x over previous
import functools

import jax
import jax.numpy as jnp
from jax import lax
from jax.experimental import pallas as pl
from jax.experimental.pallas import tpu as pltpu
from jax.experimental.pallas import tpu_sc as plsc

_ND = 13
_NF = 26
_VOCAB = 100000
_EMB = 16
_ROW = _ND + _NF * _EMB

_NC = 2
_NS = 16
_NW = _NC * _NS

_T = 128
_IDX_PER_CHUNK = _T * _NF
_IDX_ROWS = _IDX_PER_CHUNK // 128
_VOUT = _T * _ROW


def _sc_embed(n_tokens):
    chunks_per_tile = n_tokens // (_NW * _T)
    mesh = plsc.VectorSubcoreMesh(core_axis_name="c", subcore_axis_name="s")

    @functools.partial(
        pl.kernel,
        mesh=mesh,
        out_type=jax.ShapeDtypeStruct((n_tokens, _ROW), jnp.float32),
        scratch_types=[
            pltpu.VMEM((_IDX_ROWS, 128), jnp.int32),
            pltpu.VMEM((_IDX_PER_CHUNK, _EMB), jnp.float32),
            pltpu.VMEM((_T, 16), jnp.float32),
            pltpu.VMEM((_T, _ROW), jnp.float32),
            pltpu.SemaphoreType.DMA,
        ],
        compiler_params=pltpu.CompilerParams(
            use_tc_tiling_on_sc=False, needs_layout_passes=False),
    )
    def k(ftab, idx2, xdp, out, idx_v, emb_v, dense_v, v_out, sem):
        wid = lax.axis_index("s") * _NC + lax.axis_index("c")
        lanes = lax.iota(jnp.int32, 16)
        dmask = lanes < _ND

        def chunk(c, carry):
            g = wid * chunks_per_tile + c
            tok0 = g * _T
            pltpu.sync_copy(idx2.at[pl.ds(g * _IDX_ROWS, _IDX_ROWS)], idx_v)
            pltpu.sync_copy(xdp.at[pl.ds(tok0, _T)], dense_v)
            cps = [
                pltpu.async_copy(
                    ftab.at[idx_v.at[j]], emb_v.at[pl.ds(j * 128, 128)], sem)
                for j in range(_IDX_ROWS)
            ]
            for cp in cps:
                cp.wait()

            def tok(t, carry2):
                trow = jnp.full((16,), t, dtype=jnp.int32)
                plsc.store_scatter(v_out, [trow, lanes], dense_v[t], mask=dmask)
                for f in range(_NF):
                    plsc.store_scatter(
                        v_out, [trow, lanes + (_ND + f * _EMB)],
                        emb_v[t * _NF + f])
                return carry2

            lax.fori_loop(0, _T, tok, 0)
            pltpu.sync_copy(v_out, out.at[pl.ds(tok0, _T)])
            return carry

        lax.fori_loop(0, chunks_per_tile, chunk, 0)

    return k


def kernel(x, tables):
    b, seq, f_dim = x.shape
    n = b * seq
    xf = x.reshape(n, f_dim)
    idx = xf[:, _ND:].astype(jnp.int32) + (
        jnp.arange(_NF, dtype=jnp.int32) * _VOCAB)[None, :]
    idx2 = idx.reshape(n * _NF // 128, 128)
    xdp = jnp.pad(xf[:, :_ND], ((0, 0), (0, 16 - _ND)))
    ftab = tables.reshape(_NF * _VOCAB, _EMB)
    out = _sc_embed(n)(ftab, idx2, xdp)
    return out.reshape(b, seq, _ROW)


# speedup vs baseline: 8.1175x; 1.2114x over previous
import functools

import jax
import jax.numpy as jnp
from jax import lax
from jax.experimental import pallas as pl
from jax.experimental.pallas import tpu as pltpu
from jax.experimental.pallas import tpu_sc as plsc

_ND = 13
_NF = 26
_VOCAB = 100000
_EMB = 16
_ROW = _ND + _NF * _EMB

_NC = 2
_NS = 16
_NW = _NC * _NS

_BPC = 2
_SEQ = 50
_T = _BPC * _SEQ


def _sc_embed(n_batch, f_dim):
    n_chunks = n_batch // _BPC
    chunks_per_tile = n_chunks // _NW
    mesh = plsc.VectorSubcoreMesh(core_axis_name="c", subcore_axis_name="s")

    @functools.partial(
        pl.kernel,
        mesh=mesh,
        out_type=jax.ShapeDtypeStruct((n_batch, _SEQ, _ROW), jnp.float32),
        scratch_types=[
            pltpu.VMEM((_T, f_dim), jnp.float32),
            pltpu.VMEM((_NF, _T), jnp.int32),
            pltpu.VMEM((_NF * _T, _EMB), jnp.float32),
            pltpu.VMEM((_BPC, _SEQ, _ROW), jnp.float32),
            pltpu.SemaphoreType.DMA,
        ],
        compiler_params=pltpu.CompilerParams(
            use_tc_tiling_on_sc=False, needs_layout_passes=False),
    )
    def k(x3, tbl, out, x_v, idx_v, emb_v, v_out, sem):
        wid = lax.axis_index("s") * _NC + lax.axis_index("c")
        lanes = lax.iota(jnp.int32, 16)
        dmask = lanes < _ND
        himask = lanes < (_NF - 16)

        def chunk(c, carry):
            g = wid * chunks_per_tile + c
            pltpu.sync_copy(x3.at[g], x_v)

            for b2 in range(_BPC):
                bvec = jnp.full((16,), b2, dtype=jnp.int32)

                def build(s, carry2):
                    t = b2 * _SEQ + s
                    tvec = jnp.full((16,), t, dtype=jnp.int32)
                    svec = jnp.full((16,), s, dtype=jnp.int32)
                    c0 = plsc.load_gather(x_v, [tvec, lanes + _ND])
                    c1 = plsc.load_gather(
                        x_v, [tvec, lanes + (_ND + 16)], mask=himask)
                    i0 = lax.convert_element_type(c0, jnp.int32)
                    i1 = lax.convert_element_type(c1, jnp.int32)
                    plsc.store_scatter(idx_v, [lanes, tvec], i0)
                    plsc.store_scatter(
                        idx_v, [lanes + 16, tvec], i1, mask=himask)
                    dv = plsc.load_gather(x_v, [tvec, lanes], mask=dmask)
                    plsc.store_scatter(
                        v_out, [bvec, svec, lanes], dv, mask=dmask)
                    return carry2

                lax.fori_loop(0, _SEQ, build, 0)

            cps = [
                pltpu.async_copy(
                    tbl.at[f].at[idx_v.at[f]],
                    emb_v.at[pl.ds(f * _T, _T)], sem)
                for f in range(_NF)
            ]
            for cp in cps:
                cp.wait()

            for b2 in range(_BPC):
                bvec = jnp.full((16,), b2, dtype=jnp.int32)

                def weave(s, carry2):
                    t = b2 * _SEQ + s
                    svec = jnp.full((16,), s, dtype=jnp.int32)
                    for f in range(_NF):
                        plsc.store_scatter(
                            v_out, [bvec, svec, lanes + (_ND + f * _EMB)],
                            emb_v[f * _T + t])
                    return carry2

                lax.fori_loop(0, _SEQ, weave, 0)

            pltpu.sync_copy(v_out, out.at[pl.ds(g * _BPC, _BPC)])
            return carry

        lax.fori_loop(0, chunks_per_tile, chunk, 0)

    return k


def kernel(x, tables):
    b, seq, f_dim = x.shape
    n_chunks = b * seq // _T
    x3 = x.reshape(n_chunks, _T, f_dim)
    out = _sc_embed(b, f_dim)(x3, tables)
    return out
